# baseline (speedup 1.0000x reference)
import functools

import jax
import jax.numpy as jnp
from jax import lax
from jax.experimental import pallas as pl
from jax.experimental.pallas import tpu as pltpu
from jax.experimental.pallas import tpu_sc as plsc

_CAPACITY = 0.125


def _sc_matvec_make(R_total, R_tc, D, CH=64):
    R_sc = R_total - R_tc
    info = plsc.get_sparse_core_info()
    NC = info.num_cores
    NW = NC * info.num_subcores
    rpw = R_sc // NW
    nch = rpw // CH
    mesh = plsc.VectorSubcoreMesh(core_axis_name="c", subcore_axis_name="s")

    @functools.partial(
        pl.kernel, mesh=mesh,
        out_type=jax.ShapeDtypeStruct((R_sc,), jnp.float32),
        scratch_types=[
            pltpu.VMEM((CH, D), jnp.float32),
            pltpu.VMEM((D,), jnp.float32),
            pltpu.VMEM((CH,), jnp.float32),
        ],
    )
    def k(h_hbm, w_hbm, out_hbm, buf, w_v, s_v):
        wid = lax.axis_index("s") * NC + lax.axis_index("c")
        base = R_tc + wid * rpw
        pltpu.sync_copy(w_hbm, w_v)

        def chunk(ci, _):
            pltpu.sync_copy(h_hbm.at[pl.ds(base + ci * CH, CH)], buf)

            def rowblk(rb, _):
                def jstep(j, accs):
                    wv = w_v[pl.ds(j * 16, 16)]
                    return tuple(
                        accs[r] + buf[rb * 16 + r, pl.ds(j * 16, 16)] * wv
                        for r in range(16))

                accs = lax.fori_loop(
                    0, D // 16, jstep,
                    tuple(jnp.zeros((16,), jnp.float32) for _ in range(16)))
                lanes = lax.iota(jnp.int32, 16)
                res = jnp.zeros((16,), jnp.float32)
                for r in range(16):
                    a = accs[r]
                    for sh in (8, 4, 2, 1):
                        a = a + a[lanes ^ sh]
                    res = jnp.where(lanes == r, a, res)
                s_v[pl.ds(rb * 16, 16)] = res
                return 0

            lax.fori_loop(0, CH // 16, rowblk, 0)
            pltpu.sync_copy(s_v, out_hbm.at[pl.ds(wid * rpw + ci * CH, CH)])
            return 0

        lax.fori_loop(0, nch, chunk, 0)

    return k


def _select(s, k):
    B, S = s.shape
    b = jax.lax.bitcast_convert_type(s, jnp.int32)
    m = jax.lax.shift_right_arithmetic(b, 31)
    key = b ^ (m & jnp.int32(0x7FFFFFFF))
    MIN32 = jnp.int32(-(2 ** 31))

    def bit_step(i, cur_u):
        bit = jax.lax.shift_left(jnp.int32(1), 31 - i)
        trial_u = cur_u | bit
        trial_s = trial_u ^ MIN32
        cnt = jnp.sum((key >= trial_s).astype(jnp.int32), axis=1,
                      keepdims=True)
        return jnp.where(cnt >= k, trial_u, cur_u)

    cur_u = jax.lax.fori_loop(0, 32, bit_step,
                              jnp.zeros((B, 1), jnp.int32))
    T = cur_u ^ MIN32
    gt = key > T
    eq = key == T
    cnt_gt = jnp.sum(gt.astype(jnp.int32), axis=1, keepdims=True)
    need = k - cnt_gt
    idx = jax.lax.broadcasted_iota(jnp.int32, (B, S), 1)

    def idx_step(i, cur):
        trial = cur | jax.lax.shift_left(jnp.int32(1), 13 - i)
        cnt = jnp.sum((eq & (idx < trial)).astype(jnp.int32), axis=1,
                      keepdims=True)
        return jnp.where(cnt <= need, trial, cur)

    tbound = jax.lax.fori_loop(0, 14, idx_step,
                               jnp.zeros((B, 1), jnp.int32))
    return (gt | (eq & (idx < tbound))).astype(jnp.float32)


def _fused_body(k, B, G, BLK, h_ref, w_ref, scores_ref, mask_ref, acc_ref):
    bb = pl.program_id(0)
    j = pl.program_id(1)
    s = jax.lax.dot_general(
        w_ref[...], h_ref[0],
        (((1,), (1,)), ((), ())),
        preferred_element_type=jnp.float32,
    )
    scores_ref[...] = s
    for r in range(B):
        @pl.when(bb == r)
        def _():
            acc_ref[r:r + 1, pl.ds(j * BLK, BLK)] = s

    @pl.when((bb == B - 1) & (j == G - 1))
    def _():
        mask_ref[...] = _select(acc_ref[...], k)


def kernel(hidden_states, W):
    B, S, D = hidden_states.shape
    k = int(_CAPACITY * S)
    BLK = 512
    G = S // BLK
    scores, mask = pl.pallas_call(
        functools.partial(_fused_body, k, B, G, BLK),
        grid=(B, G),
        in_specs=[
            pl.BlockSpec((1, BLK, D), lambda b, j: (b, j, 0)),
            pl.BlockSpec((1, D), lambda b, j: (0, 0)),
        ],
        out_specs=[
            pl.BlockSpec((1, BLK), lambda b, j: (0, b * (S // BLK) + j)),
            pl.BlockSpec((B, S), lambda b, j: (0, 0)),
        ],
        out_shape=[
            jax.ShapeDtypeStruct((1, B * S), jnp.float32),
            jax.ShapeDtypeStruct((B, S), jnp.float32),
        ],
        scratch_shapes=[pltpu.VMEM((B, S), jnp.float32)],
    )(hidden_states, W)
    scores = scores.reshape(B, S)
    R = B * S
    scores_sc = _sc_matvec_make(R, R // 2, D)(
        hidden_states.reshape(R, D), W.reshape(D))
    contrib = jnp.concatenate(
        [jnp.zeros((R // 2,), jnp.float32), scores_sc * 1e-30]).reshape(B, S)
    scores = scores + contrib
    if k >= S:
        return (jnp.ones_like(scores), scores)
    return (mask, scores)


# speedup vs baseline: 1.7479x; 1.7479x over previous
import functools

import jax
import jax.numpy as jnp
from jax import lax
from jax.experimental import pallas as pl
from jax.experimental.pallas import tpu as pltpu
from jax.experimental.pallas import tpu_sc as plsc

_CAPACITY = 0.125


def _sc_matvec_make(R_total, R_tc, D, CH=64):
    R_sc = R_total - R_tc
    info = plsc.get_sparse_core_info()
    NC = info.num_cores
    NW = NC * info.num_subcores
    rpw = R_sc // NW
    nch = rpw // CH
    mesh = plsc.VectorSubcoreMesh(core_axis_name="c", subcore_axis_name="s")

    @functools.partial(
        pl.kernel, mesh=mesh,
        out_type=jax.ShapeDtypeStruct((R_sc,), jnp.float32),
        scratch_types=[
            pltpu.VMEM((CH, D), jnp.float32),
            pltpu.VMEM((D,), jnp.float32),
            pltpu.VMEM((CH,), jnp.float32),
        ],
    )
    def k(h_hbm, w_hbm, out_hbm, buf, w_v, s_v):
        wid = lax.axis_index("s") * NC + lax.axis_index("c")
        base = R_tc + wid * rpw
        pltpu.sync_copy(w_hbm, w_v)

        def chunk(ci, _):
            pltpu.sync_copy(h_hbm.at[pl.ds(base + ci * CH, CH)], buf)

            def rowblk(rb, _):
                def jstep(j, accs):
                    wv = w_v[pl.ds(j * 16, 16)]
                    return tuple(
                        accs[r] + buf[rb * 16 + r, pl.ds(j * 16, 16)] * wv
                        for r in range(16))

                accs = lax.fori_loop(
                    0, D // 16, jstep,
                    tuple(jnp.zeros((16,), jnp.float32) for _ in range(16)))
                lanes = lax.iota(jnp.int32, 16)
                res = jnp.zeros((16,), jnp.float32)
                for r in range(16):
                    a = accs[r]
                    for sh in (8, 4, 2, 1):
                        a = a + a[lanes ^ sh]
                    res = jnp.where(lanes == r, a, res)
                s_v[pl.ds(rb * 16, 16)] = res
                return 0

            lax.fori_loop(0, CH // 16, rowblk, 0)
            pltpu.sync_copy(s_v, out_hbm.at[pl.ds(wid * rpw + ci * CH, CH)])
            return 0

        lax.fori_loop(0, nch, chunk, 0)

    return k


def _select(s, k):
    B, S = s.shape
    b = jax.lax.bitcast_convert_type(s, jnp.int32)
    m = jax.lax.shift_right_arithmetic(b, 31)
    key = b ^ (m & jnp.int32(0x7FFFFFFF))
    MIN32 = jnp.int32(-(2 ** 31))

    def bit_step(i, cur_u):
        bit = jax.lax.shift_left(jnp.int32(1), 31 - i)
        trial_u = cur_u | bit
        trial_s = trial_u ^ MIN32
        cnt = jnp.sum((key >= trial_s).astype(jnp.int32), axis=1,
                      keepdims=True)
        return jnp.where(cnt >= k, trial_u, cur_u)

    cur_u = jax.lax.fori_loop(0, 32, bit_step,
                              jnp.zeros((B, 1), jnp.int32))
    T = cur_u ^ MIN32
    gt = key > T
    eq = key == T
    cnt_gt = jnp.sum(gt.astype(jnp.int32), axis=1, keepdims=True)
    need = k - cnt_gt
    idx = jax.lax.broadcasted_iota(jnp.int32, (B, S), 1)

    def idx_step(i, cur):
        trial = cur | jax.lax.shift_left(jnp.int32(1), 13 - i)
        cnt = jnp.sum((eq & (idx < trial)).astype(jnp.int32), axis=1,
                      keepdims=True)
        return jnp.where(cnt <= need, trial, cur)

    tbound = jax.lax.fori_loop(0, 14, idx_step,
                               jnp.zeros((B, 1), jnp.int32))
    return (gt | (eq & (idx < tbound))).astype(jnp.float32)


def _fused_body(k, B, G, BLK, h_ref, w_ref, scores_ref, mask_ref, acc_ref):
    bb = pl.program_id(0)
    j = pl.program_id(1)
    s = jax.lax.dot_general(
        w_ref[...], h_ref[0],
        (((1,), (1,)), ((), ())),
        preferred_element_type=jnp.float32,
    )
    scores_ref[...] = s
    for r in range(B):
        @pl.when(bb == r)
        def _():
            acc_ref[r:r + 1, pl.ds(j * BLK, BLK)] = s

    @pl.when((bb == B - 1) & (j == G - 1))
    def _():
        mask_ref[...] = _select(acc_ref[...], k)


def kernel(hidden_states, W):
    B, S, D = hidden_states.shape
    k = int(_CAPACITY * S)
    BLK = 1024
    G = S // BLK
    scores, mask = pl.pallas_call(
        functools.partial(_fused_body, k, B, G, BLK),
        grid=(B, G),
        in_specs=[
            pl.BlockSpec((1, BLK, D), lambda b, j: (b, j, 0)),
            pl.BlockSpec((1, D), lambda b, j: (0, 0)),
        ],
        out_specs=[
            pl.BlockSpec((1, BLK), lambda b, j: (0, b * (S // BLK) + j)),
            pl.BlockSpec((B, S), lambda b, j: (0, 0)),
        ],
        out_shape=[
            jax.ShapeDtypeStruct((1, B * S), jnp.float32),
            jax.ShapeDtypeStruct((B, S), jnp.float32),
        ],
        scratch_shapes=[pltpu.VMEM((B, S), jnp.float32)],
    )(hidden_states, W)
    scores = scores.reshape(B, S)
    if k >= S:
        return (jnp.ones_like(scores), scores)
    return (mask, scores)


# speedup vs baseline: 2.0162x; 1.1534x over previous
import functools

import jax
import jax.numpy as jnp
from jax import lax
from jax.experimental import pallas as pl
from jax.experimental.pallas import tpu as pltpu
from jax.experimental.pallas import tpu_sc as plsc

_CAPACITY = 0.125


def _sc_matvec_make(R_total, R_tc, D, CH=64):
    R_sc = R_total - R_tc
    info = plsc.get_sparse_core_info()
    NC = info.num_cores
    NW = NC * info.num_subcores
    rpw = R_sc // NW
    nch = rpw // CH
    mesh = plsc.VectorSubcoreMesh(core_axis_name="c", subcore_axis_name="s")

    @functools.partial(
        pl.kernel, mesh=mesh,
        out_type=jax.ShapeDtypeStruct((R_sc,), jnp.float32),
        scratch_types=[
            pltpu.VMEM((CH, D), jnp.float32),
            pltpu.VMEM((D,), jnp.float32),
            pltpu.VMEM((CH,), jnp.float32),
        ],
    )
    def k(h_hbm, w_hbm, out_hbm, buf, w_v, s_v):
        wid = lax.axis_index("s") * NC + lax.axis_index("c")
        base = R_tc + wid * rpw
        pltpu.sync_copy(w_hbm, w_v)

        def chunk(ci, _):
            pltpu.sync_copy(h_hbm.at[pl.ds(base + ci * CH, CH)], buf)

            def rowblk(rb, _):
                def jstep(j, accs):
                    wv = w_v[pl.ds(j * 16, 16)]
                    return tuple(
                        accs[r] + buf[rb * 16 + r, pl.ds(j * 16, 16)] * wv
                        for r in range(16))

                accs = lax.fori_loop(
                    0, D // 16, jstep,
                    tuple(jnp.zeros((16,), jnp.float32) for _ in range(16)))
                lanes = lax.iota(jnp.int32, 16)
                res = jnp.zeros((16,), jnp.float32)
                for r in range(16):
                    a = accs[r]
                    for sh in (8, 4, 2, 1):
                        a = a + a[lanes ^ sh]
                    res = jnp.where(lanes == r, a, res)
                s_v[pl.ds(rb * 16, 16)] = res
                return 0

            lax.fori_loop(0, CH // 16, rowblk, 0)
            pltpu.sync_copy(s_v, out_hbm.at[pl.ds(wid * rpw + ci * CH, CH)])
            return 0

        lax.fori_loop(0, nch, chunk, 0)

    return k


def _select(s, k):
    B, S = s.shape
    b = jax.lax.bitcast_convert_type(s, jnp.int32)
    m = jax.lax.shift_right_arithmetic(b, 31)
    key = b ^ (m & jnp.int32(0x7FFFFFFF))
    MIN32 = jnp.int32(-(2 ** 31))

    def bit_step(i, cur_u):
        bit = jax.lax.shift_left(jnp.int32(1), 31 - i)
        trial_u = cur_u | bit
        trial_s = trial_u ^ MIN32
        cnt = jnp.sum((key >= trial_s).astype(jnp.int32), axis=1,
                      keepdims=True)
        return jnp.where(cnt >= k, trial_u, cur_u)

    cur_u = jax.lax.fori_loop(0, 32, bit_step,
                              jnp.zeros((B, 1), jnp.int32))
    T = cur_u ^ MIN32
    gt = key > T
    eq = key == T
    cnt_gt = jnp.sum(gt.astype(jnp.int32), axis=1, keepdims=True)
    need = k - cnt_gt
    idx = jax.lax.broadcasted_iota(jnp.int32, (B, S), 1)

    def idx_step(i, cur):
        trial = cur | jax.lax.shift_left(jnp.int32(1), 13 - i)
        cnt = jnp.sum((eq & (idx < trial)).astype(jnp.int32), axis=1,
                      keepdims=True)
        return jnp.where(cnt <= need, trial, cur)

    tbound = jax.lax.fori_loop(0, 14, idx_step,
                               jnp.zeros((B, 1), jnp.int32))
    return (gt | (eq & (idx < tbound))).astype(jnp.float32)


def _fused_body(k, B, G, BLK, h_ref, w_ref, scores_ref, mask_ref, acc_ref):
    bb = pl.program_id(0)
    j = pl.program_id(1)
    s = jax.lax.dot_general(
        w_ref[...], h_ref[0],
        (((1,), (1,)), ((), ())),
        preferred_element_type=jnp.float32,
    )
    scores_ref[...] = s
    for r in range(B):
        @pl.when(bb == r)
        def _():
            acc_ref[r:r + 1, pl.ds(j * BLK, BLK)] = s

    @pl.when((bb == B - 1) & (j == G - 1))
    def _():
        mask_ref[...] = _select(acc_ref[...], k)


def kernel(hidden_states, W):
    B, S, D = hidden_states.shape
    k = int(_CAPACITY * S)
    BLK = 2048
    G = S // BLK
    scores, mask = pl.pallas_call(
        functools.partial(_fused_body, k, B, G, BLK),
        grid=(B, G),
        in_specs=[
            pl.BlockSpec((1, BLK, D), lambda b, j: (b, j, 0)),
            pl.BlockSpec((1, D), lambda b, j: (0, 0)),
        ],
        out_specs=[
            pl.BlockSpec((1, BLK), lambda b, j: (0, b * (S // BLK) + j)),
            pl.BlockSpec((B, S), lambda b, j: (0, 0)),
        ],
        out_shape=[
            jax.ShapeDtypeStruct((1, B * S), jnp.float32),
            jax.ShapeDtypeStruct((B, S), jnp.float32),
        ],
        scratch_shapes=[pltpu.VMEM((B, S), jnp.float32)],
    )(hidden_states, W)
    scores = scores.reshape(B, S)
    if k >= S:
        return (jnp.ones_like(scores), scores)
    return (mask, scores)
